# baseline (speedup 1.0000x reference)
import jax
import jax.numpy as jnp
from jax.experimental import pallas as pl

D = 256
NX = NY = NZ = 32
NYZ = NY * NZ
DBLK = 64


def _body(xt_ref, yt_ref, zt_ref, out_ref):
    xt = xt_ref[...]
    yt = yt_ref[...]
    zt = zt_ref[...]
    yz = yt[:, :, None] + zt[:, None, :]
    yz = yz.reshape(DBLK, 1, NYZ)
    pos = xt[:, :, None] + yz
    out_ref[...] = pos[None]


def kernel(features, x_embed, y_embed, z_embed):
    b = features.shape[0]
    xt = x_embed[:NX].T
    yt = y_embed[:NY].T
    zt = z_embed[:NZ].T
    grid = (b, D // DBLK)
    out = pl.pallas_call(
        _body,
        grid=grid,
        in_specs=[
            pl.BlockSpec((DBLK, NX), lambda bi, di: (di, 0)),
            pl.BlockSpec((DBLK, NY), lambda bi, di: (di, 0)),
            pl.BlockSpec((DBLK, NZ), lambda bi, di: (di, 0)),
        ],
        out_specs=pl.BlockSpec((1, DBLK, NX, NYZ), lambda bi, di: (bi, di, 0, 0)),
        out_shape=jax.ShapeDtypeStruct((b, D, NX, NYZ), jnp.float32),
    )(xt, yt, zt)
    return out.reshape(b, D, NX, NY, NZ)
